# baseline (speedup 1.0000x reference)
import functools

import jax
import jax.numpy as jnp
from jax import lax
from jax.experimental import pallas as pl
from jax.experimental.pallas import tpu as pltpu
from jax.experimental.pallas import tpu_sc as plsc

B = 16384
D = 64
NC, NS = 2, 16
NW = NC * NS
PER_W = B // NW
IDX_CHUNK = 128
TC_BLK = 2048


def _sc_gather6(h_i, t_i, hc_i, tc_i, r_i, ent, rnorm, rhyper):
    mesh = plsc.VectorSubcoreMesh(core_axis_name="c", subcore_axis_name="s")
    row_t = jax.ShapeDtypeStruct((B, D), jnp.float32)

    @functools.partial(
        pl.kernel,
        mesh=mesh,
        out_type=[row_t] * 6,
        scratch_types=[
            pltpu.VMEM((PER_W,), jnp.int32),
            pltpu.VMEM((PER_W, D), jnp.float32),
            pltpu.SemaphoreType.DMA,
        ],
        compiler_params=pltpu.CompilerParams(use_tc_tiling_on_sc=False),
    )
    def k(h_hbm, t_hbm, hc_hbm, tc_hbm, r_hbm, ent_hbm, rn_hbm, rh_hbm,
          head_o, tail_o, ch_o, ct_o, rn_o, rh_o, idx_v, rows_v, sem):
        wid = lax.axis_index("s") * NC + lax.axis_index("c")
        base = wid * PER_W
        for idx_hbm, tab_hbm, out_hbm in (
            (h_hbm, ent_hbm, head_o),
            (t_hbm, ent_hbm, tail_o),
            (hc_hbm, ent_hbm, ch_o),
            (tc_hbm, ent_hbm, ct_o),
            (r_hbm, rn_hbm, rn_o),
            (r_hbm, rh_hbm, rh_o),
        ):
            pltpu.sync_copy(idx_hbm.at[pl.ds(base, PER_W)], idx_v)
            copies = []
            for c in range(PER_W // IDX_CHUNK):
                copies.append(pltpu.async_copy(
                    tab_hbm.at[idx_v.at[pl.ds(c * IDX_CHUNK, IDX_CHUNK)]],
                    rows_v.at[pl.ds(c * IDX_CHUNK, IDX_CHUNK)],
                    sem,
                ))
            for cp in copies:
                cp.wait()
            pltpu.sync_copy(rows_v, out_hbm.at[pl.ds(base, PER_W)])

    return k(h_i, t_i, hc_i, tc_i, r_i, ent, rnorm, rhyper)


def _tc_loss_body(head_r, tail_r, ch_r, ct_r, rn_r, rh_r, out_r):
    i = pl.program_id(0)

    @pl.when(i == 0)
    def _():
        out_r[...] = jnp.zeros((1, 1), jnp.float32)

    hd = head_r[...]
    tl = tail_r[...]
    c_h = ch_r[...]
    c_t = ct_r[...]
    rn = rn_r[...]
    rh = rh_r[...]

    d = hd - tl
    dc = c_h - c_t
    s_pos = jnp.sum(rn * d, axis=1, keepdims=True)
    s_neg = jnp.sum(rn * dc, axis=1, keepdims=True)
    pv = d - s_pos * rn + rh + 1e-6
    nv = dc - s_neg * rn + rh + 1e-6
    pos = jnp.sqrt(jnp.sum(pv * pv, axis=1))
    neg = jnp.sqrt(jnp.sum(nv * nv, axis=1))
    total = jnp.sum(jnp.maximum(pos - neg + 1.0, 0.0))
    for x in (hd, tl, c_h, c_t):
        nrm = jnp.sqrt(jnp.sum(x * x, axis=1))
        total += jnp.sum(jnp.maximum(nrm - 1.0, 0.0))
    out_r[...] = out_r[...] + total


def _tc_loss(head, tail, ch, ct, rn, rh):
    out = pl.pallas_call(
        _tc_loss_body,
        grid=(B // TC_BLK,),
        in_specs=[pl.BlockSpec((TC_BLK, D), lambda i: (i, 0))] * 6,
        out_specs=pl.BlockSpec((1, 1), lambda i: (0, 0)),
        out_shape=jax.ShapeDtypeStruct((1, 1), jnp.float32),
    )(head, tail, ch, ct, rn, rh)
    return out[0, 0]


def kernel(current_triples, corrupted_triples, entity_embedding,
           relation_norm_embedding, relation_hyper_embedding):
    h = current_triples[:, 0]
    t = current_triples[:, 1]
    r = current_triples[:, 2]
    h_c = corrupted_triples[:, 0]
    t_c = corrupted_triples[:, 1]
    head, tail, ch, ct, rn, rh = _sc_gather6(
        h, t, h_c, t_c, r,
        entity_embedding, relation_norm_embedding, relation_hyper_embedding)
    return _tc_loss(head, tail, ch, ct, rn, rh)


# speedup vs baseline: 1.1337x; 1.1337x over previous
import functools

import jax
import jax.numpy as jnp
from jax import lax
from jax.experimental import pallas as pl
from jax.experimental.pallas import tpu as pltpu
from jax.experimental.pallas import tpu_sc as plsc

B = 16384
D = 64
DP = 2 * D
NC, NS = 2, 16
NW = NC * NS
PER_W = B // NW
IDX_CHUNK = 128
TC_BLK = 2048
NB = B // TC_BLK


def _sc_gather5(h2, t2, hc2, tc2, r, ent2, rel2):
    mesh = plsc.VectorSubcoreMesh(core_axis_name="c", subcore_axis_name="s")
    row_t = jax.ShapeDtypeStruct((B, DP), jnp.float32)

    @functools.partial(
        pl.kernel,
        mesh=mesh,
        out_type=[row_t] * 5,
        scratch_types=[
            pltpu.VMEM((PER_W,), jnp.int32),
            pltpu.VMEM((PER_W, DP), jnp.float32),
            pltpu.SemaphoreType.DMA,
        ],
        compiler_params=pltpu.CompilerParams(use_tc_tiling_on_sc=True),
    )
    def k(h_hbm, t_hbm, hc_hbm, tc_hbm, r_hbm, ent_hbm, rel_hbm,
          head_o, tail_o, ch_o, ct_o, rel_o, idx_v, rows_v, sem):
        wid = lax.axis_index("s") * NC + lax.axis_index("c")
        base = wid * PER_W
        for idx_hbm, tab_hbm, out_hbm in (
            (h_hbm, ent_hbm, head_o),
            (t_hbm, ent_hbm, tail_o),
            (hc_hbm, ent_hbm, ch_o),
            (tc_hbm, ent_hbm, ct_o),
            (r_hbm, rel_hbm, rel_o),
        ):
            pltpu.sync_copy(idx_hbm.at[pl.ds(base, PER_W)], idx_v)
            copies = []
            for c in range(PER_W // IDX_CHUNK):
                copies.append(pltpu.async_copy(
                    tab_hbm.at[idx_v.at[pl.ds(c * IDX_CHUNK, IDX_CHUNK)]],
                    rows_v.at[pl.ds(c * IDX_CHUNK, IDX_CHUNK)],
                    sem,
                ))
            for cp in copies:
                cp.wait()
            pltpu.sync_copy(rows_v, out_hbm.at[pl.ds(base, PER_W)])

    return k(h2, t2, hc2, tc2, r, ent2, rel2)


def _half(pair_block, parity_col):
    return jnp.where(parity_col == 0.0, pair_block[:, :D], pair_block[:, D:])


def _tc_loss_body(hp_r, tp_r, chp_r, ctp_r, rel_r, par_r, out_r):
    i = pl.program_id(0)

    @pl.when(i == 0)
    def _():
        out_r[...] = jnp.zeros((1, 1), jnp.float32)

    par = par_r[...]
    hd = _half(hp_r[...], par[:, 0:1])
    tl = _half(tp_r[...], par[:, 1:2])
    c_h = _half(chp_r[...], par[:, 2:3])
    c_t = _half(ctp_r[...], par[:, 3:4])
    rel = rel_r[...]
    rn = rel[:, :D]
    rh = rel[:, D:]

    d = hd - tl
    dc = c_h - c_t
    s_pos = jnp.sum(rn * d, axis=1, keepdims=True)
    s_neg = jnp.sum(rn * dc, axis=1, keepdims=True)
    pv = d - s_pos * rn + rh + 1e-6
    nv = dc - s_neg * rn + rh + 1e-6
    pos = jnp.sqrt(jnp.sum(pv * pv, axis=1))
    neg = jnp.sqrt(jnp.sum(nv * nv, axis=1))
    total = jnp.sum(jnp.maximum(pos - neg + 1.0, 0.0))
    for x in (hd, tl, c_h, c_t):
        nrm = jnp.sqrt(jnp.sum(x * x, axis=1))
        total += jnp.sum(jnp.maximum(nrm - 1.0, 0.0))
    out_r[...] = out_r[...] + total


def _tc_loss(headp, tailp, chp, ctp, rel, parities):
    out = pl.pallas_call(
        _tc_loss_body,
        grid=(NB,),
        in_specs=[pl.BlockSpec((TC_BLK, DP), lambda i: (i, 0))] * 5
        + [pl.BlockSpec((TC_BLK, 4), lambda i: (i, 0))],
        out_specs=pl.BlockSpec((1, 1), lambda i: (0, 0)),
        out_shape=jax.ShapeDtypeStruct((1, 1), jnp.float32),
    )(headp, tailp, chp, ctp, rel, parities)
    return out[0, 0]


def kernel(current_triples, corrupted_triples, entity_embedding,
           relation_norm_embedding, relation_hyper_embedding):
    h = current_triples[:, 0]
    t = current_triples[:, 1]
    r = current_triples[:, 2]
    h_c = corrupted_triples[:, 0]
    t_c = corrupted_triples[:, 1]

    ent2 = entity_embedding.reshape(50000, DP)
    rel2 = jnp.concatenate(
        [relation_norm_embedding, relation_hyper_embedding], axis=1)

    headp, tailp, chp, ctp, rel = _sc_gather5(
        h >> 1, t >> 1, h_c >> 1, t_c >> 1, r, ent2, rel2)

    parities = jnp.stack([h & 1, t & 1, h_c & 1, t_c & 1],
                         axis=1).astype(jnp.float32)
    return _tc_loss(headp, tailp, chp, ctp, rel, parities)
